# baseline (speedup 1.0000x reference)
import jax
import jax.numpy as jnp
from jax import lax
from jax.experimental import pallas as pl
from jax.experimental.pallas import tpu as pltpu
from jax.experimental.pallas import tpu_sc as plsc

H, W = 496, 432
C = 64
B, P = 4, 12000

NC, NS, L = 2, 16, 16
NW = NC * NS
RR = 48
XSPAN = W // RR
BH = 2
NTASK = BH * RR
SG = 4
CHUNK = 2400
CH = 128
NSLOT = 8
D16 = 16
MAPN = XSPAN * H
LISTN = P + L


def _sc_body(b0, y_hbm, x_hbm, featv_hbm, out_hbm,
             ybuf, xbuf, cells, pg16, cmap, imgbuf, idxbuf, featbuf,
             zlist, g0, g1, g2, g3, g4, g5, g6, g7, sem_s):
  gsems = (g0, g1, g2, g3, g4, g5, g6, g7)
  wid = lax.axis_index("s") * NC + lax.axis_index("c")
  iota = lax.iota(jnp.int32, L)
  iota_map = iota * MAPN
  zeros16 = jnp.zeros((L,), jnp.float32)

  def splat(v):
    return jnp.full((L,), v, jnp.int32)

  def build_idx(slot, chk, sg, cnt2):
    m0 = chk * CH

    def ib(qi, _):
      sl = m0 + qi * L
      pv = pg16[pl.ds(sl, L)]
      act = (sl + iota) < cnt2
      idxbuf[pl.ds(slot * CH + qi * L, L)] = jnp.where(act, pv + sg, 0)
      return 0

    lax.fori_loop(0, CH // L, ib, 0)

  def fire(slot, sem, cnt2=None, chk=None, sg=None):
    if cnt2 is not None:
      build_idx(slot, chk, sg, cnt2)
    return pltpu.async_copy(
        featv_hbm.at[idxbuf.at[pl.ds(slot * CH, CH)]],
        featbuf.at[slot], sem)

  def drain_gather(slot, sem):
    pltpu.make_async_copy(
        featv_hbm.at[idxbuf.at[pl.ds(slot * CH, CH)]],
        featbuf.at[slot], sem).wait()

  def drain_streams():
    pltpu.make_async_copy(
        imgbuf, out_hbm.at[pl.ds(0, D16 * MAPN)], sem_s).wait()

  def run_task(t, zcnt):
    task = t * NW + wid
    b = b0 + task // RR
    rr = task % RR
    r0 = rr * XSPAN

    def chunk_body(ci, cntv):
      base = ci * CHUNK
      pltpu.sync_copy(y_hbm.at[b, pl.ds(base, CHUNK)], ybuf)
      pltpu.sync_copy(x_hbm.at[b, pl.ds(base, CHUNK)], xbuf)
      pgb = (b * P + base + iota) * SG

      def grp(gi, cntv):
        o = gi * (2 * L)
        xv0 = xbuf[pl.ds(o, L)]
        yv0 = ybuf[pl.ds(o, L)]
        xv1 = xbuf[pl.ds(o + L, L)]
        yv1 = ybuf[pl.ds(o + L, L)]
        ux0 = xv0 - r0
        ux1 = xv1 - r0
        m0 = (ux0.astype(jnp.uint32) < XSPAN) & (yv0.astype(jnp.uint32) < H)
        m1 = (ux1.astype(jnp.uint32) < XSPAN) & (yv1.astype(jnp.uint32) < H)
        cs0 = plsc.cumsum(m0.astype(jnp.int32))
        cs1 = plsc.cumsum(m1.astype(jnp.int32))
        pc0 = plsc.all_reduce_population_count(m0)
        pc1 = plsc.all_reduce_population_count(m1)
        idx0 = cntv + cs0 - 1
        cnt1 = cntv + pc0
        idx1 = cnt1 + cs1 - 1
        plsc.store_scatter(cells, [idx0], ux0 * H + yv0, mask=m0)
        plsc.store_scatter(pg16, [idx0], pgb + o * SG, mask=m0)
        plsc.store_scatter(cells, [idx1], ux1 * H + yv1, mask=m1)
        plsc.store_scatter(pg16, [idx1], pgb + (o + L) * SG, mask=m1)
        return cnt1 + pc1

      return lax.fori_loop(0, CHUNK // (2 * L), grp, cntv)

    cntv = lax.fori_loop(0, P // CHUNK, chunk_body, splat(0))
    cnt = jnp.max(cntv)

    def mi(i, _):
      cmap[pl.ds(i * L, L)] = jnp.full((L,), -1, jnp.int32)
      return 0

    lax.fori_loop(0, MAPN // L, mi, 0)

    ngrp = (cnt + L - 1) // L

    def dgrp(gi, _):
      sl = gi * L
      cvec = cells[pl.ds(sl, L)]
      slot = sl + iota
      act = (slot < cnt).astype(jnp.int32)

      def cond(pend):
        return jnp.max(pend) > 0

      def body(pend):
        pm = pend > 0
        plsc.store_scatter(cmap, [cvec], slot, mask=pm)
        w = plsc.load_gather(cmap, [cvec], mask=pm)
        return (pm & (w < slot)).astype(jnp.int32)

      lax.while_loop(cond, body, act)
      return 0

    lax.fori_loop(0, ngrp, dgrp, 0)

    def kgrp(gi, cnt2v):
      sl = gi * L
      cvec = cells[pl.ds(sl, L)]
      pvec = pg16[pl.ds(sl, L)]
      slot = sl + iota
      act = slot < cnt
      w = plsc.load_gather(cmap, [cvec], mask=act)
      keep = act & (w == slot)
      csum = plsc.cumsum(keep.astype(jnp.int32))
      idx = cnt2v + csum - 1
      plsc.store_scatter(cells, [idx], cvec, mask=keep)
      plsc.store_scatter(pg16, [idx], pvec, mask=keep)
      return cnt2v + plsc.all_reduce_population_count(keep)

    cnt2 = jnp.max(lax.fori_loop(0, ngrp, kgrp, splat(0)))
    nch = (cnt2 + CH - 1) // CH

    for k in range(NSLOT):
      sgi, chk = k // 2, k % 2

      @pl.when(chk < nch)
      def _(k=k, sgi=sgi, chk=chk):
        fire(k, gsems[k], cnt2=cnt2, chk=chk, sg=sgi)

    @pl.when(t > 0)
    def _():
      drain_streams()

    def zq(q, _):
      cellv = plsc.load_gather(zlist, [splat(q)])
      plsc.store_scatter(imgbuf, [cellv + iota_map], zeros16)
      return 0

    lax.fori_loop(0, zcnt, zq, 0)

    for sgi in range(SG):
      if sgi > 0:
        drain_streams()

      def chunk_loop(ch, _, sgi=sgi):
        par = ch & 1
        refire = ch >= 2

        for parb in range(2):
          slot = sgi * 2 + parb

          @pl.when(refire & (par == parb))
          def _(slot=slot):
            fire(slot, gsems[slot], cnt2=cnt2, chk=ch, sg=sgi).wait()

          @pl.when((~refire) & (par == parb))
          def _(slot=slot):
            drain_gather(slot, gsems[slot])

        slotv = sgi * 2 + par
        mcnt = jnp.minimum(CH, cnt2 - ch * CH)
        slots = splat(slotv)

        def sq(q, _):
          cellv = plsc.load_gather(cells, [splat(ch * CH + q)])
          vals = plsc.load_gather(featbuf, [slots, splat(q), iota])
          plsc.store_scatter(imgbuf, [cellv + iota_map], vals)
          return 0

        lax.fori_loop(0, mcnt, sq, 0)
        return 0

      lax.fori_loop(0, nch, chunk_loop, 0)

      for cc in range(D16):
        obase = (((b - b0) * C + sgi * D16 + cc) * W + r0) * H
        pltpu.async_copy(imgbuf.at[pl.ds(cc * MAPN, MAPN)],
                         out_hbm.at[pl.ds(obase, MAPN)], sem_s)

    def zc(i, _):
      zlist[pl.ds(i * L, L)] = cells[pl.ds(i * L, L)]
      return 0

    lax.fori_loop(0, (cnt2 + L - 1) // L, zc, 0)
    return cnt2

  def zi(i, _):
    imgbuf[pl.ds(i * L, L)] = zeros16
    return 0

  lax.fori_loop(0, D16 * MAPN // L, zi, 0)
  lax.fori_loop(0, NTASK // NW, run_task, jnp.int32(0))
  drain_streams()


@jax.jit
def kernel(pillar_features, coords):
  import functools
  y32 = coords[:, :, 1].astype(jnp.int32)
  x32 = coords[:, :, 2].astype(jnp.int32)
  featv = pillar_features.reshape(B * P * (C // D16), D16)

  mesh = plsc.VectorSubcoreMesh(core_axis_name="c", subcore_axis_name="s",
                                num_cores=NC, num_subcores=NS)
  def make(b0):
    return pl.kernel(
      functools.partial(_sc_body, b0),
      out_type=jax.ShapeDtypeStruct((BH * C * H * W,), jnp.float32),
      mesh=mesh,
      compiler_params=pltpu.CompilerParams(use_tc_tiling_on_sc=False,
                                           needs_layout_passes=False),
      scratch_types=[
          pltpu.VMEM((CHUNK,), jnp.int32),
          pltpu.VMEM((CHUNK,), jnp.int32),
          pltpu.VMEM((LISTN,), jnp.int32),
          pltpu.VMEM((LISTN,), jnp.int32),
          pltpu.VMEM((MAPN,), jnp.int32),
          pltpu.VMEM((D16 * MAPN,), jnp.float32),
          pltpu.VMEM((NSLOT * CH,), jnp.int32),
          pltpu.VMEM((NSLOT, CH, D16), jnp.float32),
          pltpu.VMEM((MAPN,), jnp.int32),
          pltpu.SemaphoreType.DMA,
          pltpu.SemaphoreType.DMA,
          pltpu.SemaphoreType.DMA,
          pltpu.SemaphoreType.DMA,
          pltpu.SemaphoreType.DMA,
          pltpu.SemaphoreType.DMA,
          pltpu.SemaphoreType.DMA,
          pltpu.SemaphoreType.DMA,
          pltpu.SemaphoreType.DMA,
      ],
    )
  halves = [make(b0)(y32, x32, featv).reshape(BH, C, W, H).transpose(0, 1, 3, 2)
            for b0 in (0, BH)]
  return jnp.concatenate(halves, axis=0)


# speedup vs baseline: 1.1980x; 1.1980x over previous
import jax
import jax.numpy as jnp
from jax import lax
from jax.experimental import pallas as pl
from jax.experimental.pallas import tpu as pltpu
from jax.experimental.pallas import tpu_sc as plsc

H, W = 496, 432
C = 64
B, P = 4, 12000

NC, NS, L = 2, 16, 16
NW = NC * NS
RR = 48
XSPAN = W // RR
NTASK = B * RR
SG = 4
CHUNK = 2400
CH = 128
NSLOT = 8
D16 = 16
MAPN = XSPAN * H
LISTN = P + L


def _sc_body(y_hbm, x_hbm, featv_hbm, out_hbm,
             ybuf, xbuf, cells, pg16, cmap, imgbuf, idxbuf, featbuf,
             zlist, g0, g1, g2, g3, g4, g5, g6, g7, sem_s):
  gsems = (g0, g1, g2, g3, g4, g5, g6, g7)
  wid = lax.axis_index("s") * NC + lax.axis_index("c")
  iota = lax.iota(jnp.int32, L)
  iota_map = iota * MAPN
  zeros16 = jnp.zeros((L,), jnp.float32)

  def splat(v):
    return jnp.full((L,), v, jnp.int32)

  def build_idx(slot, chk, sg, cnt2):
    m0 = chk * CH

    def ib(qi, _):
      sl = m0 + qi * L
      pv = pg16[pl.ds(sl, L)]
      act = (sl + iota) < cnt2
      idxbuf[pl.ds(slot * CH + qi * L, L)] = jnp.where(act, pv + sg, 0)
      return 0

    lax.fori_loop(0, CH // L, ib, 0)

  def fire(slot, sem, cnt2=None, chk=None, sg=None):
    if cnt2 is not None:
      build_idx(slot, chk, sg, cnt2)
    return pltpu.async_copy(
        featv_hbm.at[idxbuf.at[pl.ds(slot * CH, CH)]],
        featbuf.at[slot], sem)

  def drain_gather(slot, sem):
    pltpu.make_async_copy(
        featv_hbm.at[idxbuf.at[pl.ds(slot * CH, CH)]],
        featbuf.at[slot], sem).wait()

  def drain_streams():
    pltpu.make_async_copy(
        imgbuf, out_hbm.at[pl.ds(0, D16 * MAPN)], sem_s).wait()

  def run_task(t, zcnt):
    task = t * NW + wid
    b = task // RR
    rr = task % RR
    r0 = rr * XSPAN

    def chunk_body(ci, cntv):
      base = ci * CHUNK
      pltpu.sync_copy(y_hbm.at[b, pl.ds(base, CHUNK)], ybuf)
      pltpu.sync_copy(x_hbm.at[b, pl.ds(base, CHUNK)], xbuf)
      pgb = (b * P + base + iota) * SG

      def grp(gi, cntv):
        o = gi * (2 * L)
        xv0 = xbuf[pl.ds(o, L)]
        yv0 = ybuf[pl.ds(o, L)]
        xv1 = xbuf[pl.ds(o + L, L)]
        yv1 = ybuf[pl.ds(o + L, L)]
        ux0 = xv0 - r0
        ux1 = xv1 - r0
        m0 = (ux0.astype(jnp.uint32) < XSPAN) & (yv0.astype(jnp.uint32) < H)
        m1 = (ux1.astype(jnp.uint32) < XSPAN) & (yv1.astype(jnp.uint32) < H)
        cs0 = plsc.cumsum(m0.astype(jnp.int32))
        cs1 = plsc.cumsum(m1.astype(jnp.int32))
        pc0 = plsc.all_reduce_population_count(m0)
        pc1 = plsc.all_reduce_population_count(m1)
        idx0 = cntv + cs0 - 1
        cnt1 = cntv + pc0
        idx1 = cnt1 + cs1 - 1
        plsc.store_scatter(cells, [idx0], ux0 * H + yv0, mask=m0)
        plsc.store_scatter(pg16, [idx0], pgb + o * SG, mask=m0)
        plsc.store_scatter(cells, [idx1], ux1 * H + yv1, mask=m1)
        plsc.store_scatter(pg16, [idx1], pgb + (o + L) * SG, mask=m1)
        return cnt1 + pc1

      return lax.fori_loop(0, CHUNK // (2 * L), grp, cntv)

    cntv = lax.fori_loop(0, P // CHUNK, chunk_body, splat(0))
    cnt = jnp.max(cntv)

    def mi(i, _):
      cmap[pl.ds(i * L, L)] = jnp.full((L,), -1, jnp.int32)
      return 0

    lax.fori_loop(0, MAPN // L, mi, 0)

    ngrp = (cnt + L - 1) // L

    def dgrp(gi, _):
      sl = gi * L
      cvec = cells[pl.ds(sl, L)]
      slot = sl + iota
      act = (slot < cnt).astype(jnp.int32)

      def cond(pend):
        return jnp.max(pend) > 0

      def body(pend):
        pm = pend > 0
        plsc.store_scatter(cmap, [cvec], slot, mask=pm)
        w = plsc.load_gather(cmap, [cvec], mask=pm)
        return (pm & (w < slot)).astype(jnp.int32)

      lax.while_loop(cond, body, act)
      return 0

    lax.fori_loop(0, ngrp, dgrp, 0)

    def kgrp(gi, cnt2v):
      sl = gi * L
      cvec = cells[pl.ds(sl, L)]
      pvec = pg16[pl.ds(sl, L)]
      slot = sl + iota
      act = slot < cnt
      w = plsc.load_gather(cmap, [cvec], mask=act)
      keep = act & (w == slot)
      csum = plsc.cumsum(keep.astype(jnp.int32))
      idx = cnt2v + csum - 1
      plsc.store_scatter(cells, [idx], cvec, mask=keep)
      plsc.store_scatter(pg16, [idx], pvec, mask=keep)
      return cnt2v + plsc.all_reduce_population_count(keep)

    cnt2 = jnp.max(lax.fori_loop(0, ngrp, kgrp, splat(0)))
    nch = (cnt2 + CH - 1) // CH

    for k in range(NSLOT):
      sgi, chk = k // 2, k % 2

      @pl.when(chk < nch)
      def _(k=k, sgi=sgi, chk=chk):
        fire(k, gsems[k], cnt2=cnt2, chk=chk, sg=sgi)

    @pl.when(t > 0)
    def _():
      drain_streams()

    def zq(q, _):
      cellv = plsc.load_gather(zlist, [splat(q)])
      plsc.store_scatter(imgbuf, [cellv + iota_map], zeros16)
      return 0

    lax.fori_loop(0, zcnt, zq, 0)

    for sgi in range(SG):
      if sgi > 0:
        drain_streams()

      def chunk_loop(ch, _, sgi=sgi):
        par = ch & 1
        refire = ch >= 2

        for parb in range(2):
          slot = sgi * 2 + parb

          @pl.when(refire & (par == parb))
          def _(slot=slot):
            fire(slot, gsems[slot], cnt2=cnt2, chk=ch, sg=sgi).wait()

          @pl.when((~refire) & (par == parb))
          def _(slot=slot):
            drain_gather(slot, gsems[slot])

        slotv = sgi * 2 + par
        mcnt = jnp.minimum(CH, cnt2 - ch * CH)
        slots = splat(slotv)

        def sq(q, _):
          cellv = plsc.load_gather(cells, [splat(ch * CH + q)])
          vals = plsc.load_gather(featbuf, [slots, splat(q), iota])
          plsc.store_scatter(imgbuf, [cellv + iota_map], vals)
          return 0

        lax.fori_loop(0, mcnt, sq, 0)
        return 0

      lax.fori_loop(0, nch, chunk_loop, 0)

      for cc in range(D16):
        obase = ((b * C + sgi * D16 + cc) * W + r0) * H
        pltpu.async_copy(imgbuf.at[pl.ds(cc * MAPN, MAPN)],
                         out_hbm.at[pl.ds(obase, MAPN)], sem_s)

    def zc(i, _):
      zlist[pl.ds(i * L, L)] = cells[pl.ds(i * L, L)]
      return 0

    lax.fori_loop(0, (cnt2 + L - 1) // L, zc, 0)
    return cnt2

  def zi(i, _):
    imgbuf[pl.ds(i * L, L)] = zeros16
    return 0

  lax.fori_loop(0, D16 * MAPN // L, zi, 0)
  lax.fori_loop(0, NTASK // NW, run_task, jnp.int32(0))
  drain_streams()


@jax.jit
def kernel(pillar_features, coords):
  y32 = coords[:, :, 1].astype(jnp.int32)
  x32 = coords[:, :, 2].astype(jnp.int32)
  featv = pillar_features.reshape(B * P * (C // D16), D16)

  mesh = plsc.VectorSubcoreMesh(core_axis_name="c", subcore_axis_name="s",
                                num_cores=NC, num_subcores=NS)
  f = pl.kernel(
      _sc_body,
      out_type=jax.ShapeDtypeStruct((B * C * H * W,), jnp.float32),
      mesh=mesh,
      compiler_params=pltpu.CompilerParams(use_tc_tiling_on_sc=False,
                                           needs_layout_passes=False),
      scratch_types=[
          pltpu.VMEM((CHUNK,), jnp.int32),
          pltpu.VMEM((CHUNK,), jnp.int32),
          pltpu.VMEM((LISTN,), jnp.int32),
          pltpu.VMEM((LISTN,), jnp.int32),
          pltpu.VMEM((MAPN,), jnp.int32),
          pltpu.VMEM((D16 * MAPN,), jnp.float32),
          pltpu.VMEM((NSLOT * CH,), jnp.int32),
          pltpu.VMEM((NSLOT, CH, D16), jnp.float32),
          pltpu.VMEM((MAPN,), jnp.int32),
          pltpu.SemaphoreType.DMA,
          pltpu.SemaphoreType.DMA,
          pltpu.SemaphoreType.DMA,
          pltpu.SemaphoreType.DMA,
          pltpu.SemaphoreType.DMA,
          pltpu.SemaphoreType.DMA,
          pltpu.SemaphoreType.DMA,
          pltpu.SemaphoreType.DMA,
          pltpu.SemaphoreType.DMA,
      ],
  )
  out_flat = f(y32, x32, featv)
  return out_flat.reshape(B, C, W, H).transpose(0, 1, 3, 2)
